# Initial kernel scaffold; baseline (speedup 1.0000x reference)
import jax
import jax.numpy as jnp
from jax.experimental import pallas as pl


def kernel(x, batch, gamma, beta, alpha):
    raise NotImplementedError("write your pallas kernel here")


# speedup vs baseline: 11.0955x; 11.0955x over previous
import jax
import jax.numpy as jnp
from jax.experimental import pallas as pl

_G = 64


def _stats_kernel(b_ref, x_ref, sum_ref, sq_ref, ct_ref):
    i = pl.program_id(0)

    @pl.when(i == 0)
    def _():
        sum_ref[...] = jnp.zeros_like(sum_ref)
        sq_ref[...] = jnp.zeros_like(sq_ref)
        ct_ref[...] = jnp.zeros_like(ct_ref)

    bb = b_ref[0, 0, :]
    xb = x_ref[...]
    onehot = (bb[:, None] == jax.lax.broadcasted_iota(
        jnp.int32, (bb.shape[0], _G), 1)).astype(jnp.float32)
    dn = (((0,), (0,)), ((), ()))
    sum_ref[...] += jax.lax.dot_general(
        onehot, xb, dn, preferred_element_type=jnp.float32)
    sq_ref[...] += jax.lax.dot_general(
        onehot, xb * xb, dn, preferred_element_type=jnp.float32)
    ct_ref[0, :] += jnp.sum(onehot, axis=0)


def _norm_kernel(b_ref, x_ref, sum_ref, sq_ref, ct_ref, g_ref, be_ref,
                 al_ref, out_ref):
    bb = b_ref[0, 0, :]
    xb = x_ref[...]
    ct = jnp.maximum(ct_ref[0, :], 1.0)
    inv_ct = (1.0 / ct)[:, None]
    mean = sum_ref[...] * inv_ct
    meansq = sq_ref[...] * inv_ct
    al = al_ref[0, 0]
    v = meansq - (al * (2.0 - al)) * mean * mean
    v = jnp.maximum(v, 0.0)
    scale = g_ref[...] / (jnp.sqrt(v) + 1e-5)
    bias = be_ref[...] - scale * (al * mean)
    onehot = (bb[:, None] == jax.lax.broadcasted_iota(
        jnp.int32, (bb.shape[0], _G), 1)).astype(jnp.float32)
    dn = (((1,), (0,)), ((), ()))
    sc_r = jax.lax.dot_general(
        onehot, scale, dn, preferred_element_type=jnp.float32)
    bi_r = jax.lax.dot_general(
        onehot, bias, dn, preferred_element_type=jnp.float32)
    out_ref[...] = sc_r * xb + bi_r


def kernel(x, batch, gamma, beta, alpha):
    n, hd = x.shape
    bn = 2000
    grid = n // bn
    batch3 = batch.astype(jnp.int32).reshape(grid, 1, bn)
    gamma2 = gamma.reshape(1, hd)
    beta2 = beta.reshape(1, hd)
    alpha2 = alpha.reshape(1, 1)

    b_spec = pl.BlockSpec((1, 1, bn), lambda i: (i, 0, 0))
    x_spec = pl.BlockSpec((bn, hd), lambda i: (i, 0))
    g_spec = pl.BlockSpec((_G, hd), lambda i: (0, 0))
    ct_spec = pl.BlockSpec((1, _G), lambda i: (0, 0))

    sums, sqs, cts = pl.pallas_call(
        _stats_kernel,
        grid=(grid,),
        in_specs=[b_spec, x_spec],
        out_specs=[g_spec, g_spec, ct_spec],
        out_shape=[
            jax.ShapeDtypeStruct((_G, hd), jnp.float32),
            jax.ShapeDtypeStruct((_G, hd), jnp.float32),
            jax.ShapeDtypeStruct((1, _G), jnp.float32),
        ],
    )(batch3, x)

    out = pl.pallas_call(
        _norm_kernel,
        grid=(grid,),
        in_specs=[b_spec, x_spec, g_spec, g_spec, ct_spec,
                  pl.BlockSpec((1, hd), lambda i: (0, 0)),
                  pl.BlockSpec((1, hd), lambda i: (0, 0)),
                  pl.BlockSpec((1, 1), lambda i: (0, 0))],
        out_specs=x_spec,
        out_shape=jax.ShapeDtypeStruct((n, hd), jnp.float32),
    )(batch3, x, sums, sqs, cts, gamma2, beta2, alpha2)
    return out


# speedup vs baseline: 12.2749x; 1.1063x over previous
import jax
import jax.numpy as jnp
from jax.experimental import pallas as pl

_G = 64


def _stats_kernel(b_ref, x_ref, sum_ref, sq_ref, ct_ref):
    i = pl.program_id(0)

    @pl.when(i == 0)
    def _():
        sum_ref[...] = jnp.zeros_like(sum_ref)
        sq_ref[...] = jnp.zeros_like(sq_ref)
        ct_ref[...] = jnp.zeros_like(ct_ref)

    bb = b_ref[0, 0, :]
    xb = x_ref[...]
    onehot = (bb[:, None] == jax.lax.broadcasted_iota(
        jnp.int32, (bb.shape[0], _G), 1)).astype(jnp.float32)
    dn = (((0,), (0,)), ((), ()))
    sum_ref[...] += jax.lax.dot_general(
        onehot, xb, dn, preferred_element_type=jnp.float32)
    sq_ref[...] += jax.lax.dot_general(
        onehot, xb * xb, dn, preferred_element_type=jnp.float32)
    ct_ref[0, :] += jnp.sum(onehot, axis=0)


def _norm_kernel(b_ref, x_ref, sum_ref, sq_ref, ct_ref, g_ref, be_ref,
                 al_ref, out_ref):
    bb = b_ref[0, 0, :]
    xb = x_ref[...]
    ct = jnp.maximum(ct_ref[0, :], 1.0)
    inv_ct = (1.0 / ct)[:, None]
    mean = sum_ref[...] * inv_ct
    meansq = sq_ref[...] * inv_ct
    al = al_ref[0, 0]
    v = meansq - (al * (2.0 - al)) * mean * mean
    v = jnp.maximum(v, 0.0)
    scale = g_ref[...] / (jnp.sqrt(v) + 1e-5)
    bias = be_ref[...] - scale * (al * mean)
    onehot = (bb[:, None] == jax.lax.broadcasted_iota(
        jnp.int32, (bb.shape[0], _G), 1)).astype(jnp.float32)
    dn = (((1,), (0,)), ((), ()))
    sc_r = jax.lax.dot_general(
        onehot, scale, dn, preferred_element_type=jnp.float32)
    bi_r = jax.lax.dot_general(
        onehot, bias, dn, preferred_element_type=jnp.float32)
    out_ref[...] = sc_r * xb + bi_r


def kernel(x, batch, gamma, beta, alpha):
    n, hd = x.shape
    bn = 4000
    grid = n // bn
    batch3 = batch.astype(jnp.int32).reshape(grid, 1, bn)
    gamma2 = gamma.reshape(1, hd)
    beta2 = beta.reshape(1, hd)
    alpha2 = alpha.reshape(1, 1)

    b_spec = pl.BlockSpec((1, 1, bn), lambda i: (i, 0, 0))
    x_spec = pl.BlockSpec((bn, hd), lambda i: (i, 0))
    g_spec = pl.BlockSpec((_G, hd), lambda i: (0, 0))
    ct_spec = pl.BlockSpec((1, _G), lambda i: (0, 0))

    sums, sqs, cts = pl.pallas_call(
        _stats_kernel,
        grid=(grid,),
        in_specs=[b_spec, x_spec],
        out_specs=[g_spec, g_spec, ct_spec],
        out_shape=[
            jax.ShapeDtypeStruct((_G, hd), jnp.float32),
            jax.ShapeDtypeStruct((_G, hd), jnp.float32),
            jax.ShapeDtypeStruct((1, _G), jnp.float32),
        ],
    )(batch3, x)

    out = pl.pallas_call(
        _norm_kernel,
        grid=(grid,),
        in_specs=[b_spec, x_spec, g_spec, g_spec, ct_spec,
                  pl.BlockSpec((1, hd), lambda i: (0, 0)),
                  pl.BlockSpec((1, hd), lambda i: (0, 0)),
                  pl.BlockSpec((1, 1), lambda i: (0, 0))],
        out_specs=x_spec,
        out_shape=jax.ShapeDtypeStruct((n, hd), jnp.float32),
    )(batch3, x, sums, sqs, cts, gamma2, beta2, alpha2)
    return out


# speedup vs baseline: 12.5212x; 1.0201x over previous
import jax
import jax.numpy as jnp
from jax.experimental import pallas as pl

_G = 64


def _stats_kernel(b_ref, x_ref, sum_ref, sq_ref, ct_ref):
    i = pl.program_id(0)

    @pl.when(i == 0)
    def _():
        sum_ref[...] = jnp.zeros_like(sum_ref)
        sq_ref[...] = jnp.zeros_like(sq_ref)
        ct_ref[...] = jnp.zeros_like(ct_ref)

    bb = b_ref[0, 0, :]
    xb = x_ref[...]
    onehot = (bb[:, None] == jax.lax.broadcasted_iota(
        jnp.int32, (bb.shape[0], _G), 1)).astype(jnp.float32)
    dn = (((0,), (0,)), ((), ()))
    sum_ref[...] += jax.lax.dot_general(
        onehot, xb, dn, preferred_element_type=jnp.float32)
    sq_ref[...] += jax.lax.dot_general(
        onehot, xb * xb, dn, preferred_element_type=jnp.float32)
    ct_ref[0, :] += jnp.sum(onehot, axis=0)


def _norm_kernel(b_ref, x_ref, sum_ref, sq_ref, ct_ref, g_ref, be_ref,
                 al_ref, out_ref):
    bb = b_ref[0, 0, :]
    xb = x_ref[...]
    ct = jnp.maximum(ct_ref[0, :], 1.0)
    inv_ct = (1.0 / ct)[:, None]
    mean = sum_ref[...] * inv_ct
    meansq = sq_ref[...] * inv_ct
    al = al_ref[0, 0]
    v = meansq - (al * (2.0 - al)) * mean * mean
    v = jnp.maximum(v, 0.0)
    scale = g_ref[...] / (jnp.sqrt(v) + 1e-5)
    bias = be_ref[...] - scale * (al * mean)
    onehot = (bb[:, None] == jax.lax.broadcasted_iota(
        jnp.int32, (bb.shape[0], _G), 1)).astype(jnp.float32)
    dn = (((1,), (0,)), ((), ()))
    sc_r = jax.lax.dot_general(
        onehot, scale, dn, preferred_element_type=jnp.float32)
    bi_r = jax.lax.dot_general(
        onehot, bias, dn, preferred_element_type=jnp.float32)
    out_ref[...] = sc_r * xb + bi_r


def kernel(x, batch, gamma, beta, alpha):
    n, hd = x.shape
    bn = 5000
    grid = n // bn
    batch3 = batch.astype(jnp.int32).reshape(grid, 1, bn)
    gamma2 = gamma.reshape(1, hd)
    beta2 = beta.reshape(1, hd)
    alpha2 = alpha.reshape(1, 1)

    b_spec = pl.BlockSpec((1, 1, bn), lambda i: (i, 0, 0))
    x_spec = pl.BlockSpec((bn, hd), lambda i: (i, 0))
    g_spec = pl.BlockSpec((_G, hd), lambda i: (0, 0))
    ct_spec = pl.BlockSpec((1, _G), lambda i: (0, 0))

    sums, sqs, cts = pl.pallas_call(
        _stats_kernel,
        grid=(grid,),
        in_specs=[b_spec, x_spec],
        out_specs=[g_spec, g_spec, ct_spec],
        out_shape=[
            jax.ShapeDtypeStruct((_G, hd), jnp.float32),
            jax.ShapeDtypeStruct((_G, hd), jnp.float32),
            jax.ShapeDtypeStruct((1, _G), jnp.float32),
        ],
    )(batch3, x)

    out = pl.pallas_call(
        _norm_kernel,
        grid=(grid,),
        in_specs=[b_spec, x_spec, g_spec, g_spec, ct_spec,
                  pl.BlockSpec((1, hd), lambda i: (0, 0)),
                  pl.BlockSpec((1, hd), lambda i: (0, 0)),
                  pl.BlockSpec((1, 1), lambda i: (0, 0))],
        out_specs=x_spec,
        out_shape=jax.ShapeDtypeStruct((n, hd), jnp.float32),
    )(batch3, x, sums, sqs, cts, gamma2, beta2, alpha2)
    return out
